# Initial kernel scaffold; baseline (speedup 1.0000x reference)
import jax
import jax.numpy as jnp
from jax.experimental import pallas as pl


def kernel(grid_x, grid_edge_index, grid_batch, surf_x, surf_edge_index, surf_batch, Wg1, bg1, Wg2, bg2, Ws1l, Ws1r, bs1, Ws2l, Ws2r, bs2):
    raise NotImplementedError("write your pallas kernel here")


# speedup vs baseline: 4.6561x; 4.6561x over previous
import functools

import jax
import jax.numpy as jnp
from jax import lax
from jax.experimental import pallas as pl
from jax.experimental.pallas import tpu as pltpu
from jax.experimental.pallas import tpu_sc as plsc

N = 10000
E = 160000
D_IN = 512
H = 128
G = 8

NC = 2
NS = 16
K = 128

EP = 163840
CPT = EP // (NC * NS * K)
CPC = EP // (NS * K)
NPAD = 10112
RPT = NPAD // NS
ORPT = 624

RB = 1000
NB = N // RB

f32 = jnp.float32


def _sc_mesh():
    return plsc.VectorSubcoreMesh(core_axis_name="c", subcore_axis_name="s")


def _fill(buf, nrows, ncols, val):
    nv = ncols // 16

    def body(i, carry):
        buf[i // nv, pl.ds((i % nv) * 16, 16)] = jnp.full((16,), val, f32)
        return carry

    lax.fori_loop(0, nrows * nv, body, 0)


def _write_out(acc, out_hbm, c, s):
    pltpu.sync_copy(acc.at[pl.ds(s * ORPT, ORPT)], out_hbm.at[c, pl.ds(s * ORPT, ORPT)])

    @pl.when(s == NS - 1)
    def _tail():
        pltpu.sync_copy(acc.at[pl.ds(NS * ORPT, N - NS * ORPT)],
                        out_hbm.at[c, pl.ds(NS * ORPT, N - NS * ORPT)])


def _zero_acc_slice(zbuf, acc, r0, width):
    del width
    for i in range(RPT // K):
        pltpu.sync_copy(zbuf, acc.at[pl.ds(r0 + i * K, K)])
    rem = RPT - (RPT // K) * K
    pltpu.sync_copy(zbuf.at[pl.ds(0, rem)], acc.at[pl.ds(r0 + (RPT // K) * K, rem)])


@functools.partial(
    pl.kernel,
    out_type=jax.ShapeDtypeStruct((NC, N, 16), f32),
    mesh=_sc_mesh(),
    scratch_types=[
        pltpu.VMEM((CPC, K), jnp.int32),
        pltpu.VMEM((K, 16), f32),
        pltpu.VMEM((K, 16), f32),
        pltpu.VMEM_SHARED((NPAD, 16), f32),
        pltpu.SemaphoreType.DMA,
    ],
)
def _sc_counts(dst_hbm, out_hbm, dst_v, ones_v, zeros_v, acc, sem):
    c = lax.axis_index("c")
    s = lax.axis_index("s")
    pltpu.sync_copy(dst_hbm.at[c, pl.ds(s * CPC, CPC)], dst_v)
    _fill(ones_v, K, 16, 1.0)
    _fill(zeros_v, K, 16, 0.0)
    _zero_acc_slice(zeros_v, acc, s * RPT, 16)
    plsc.subcore_barrier()
    for j0 in range(0, CPC, 16):
        ds = []
        for j in range(j0, j0 + 16):
            ds.append(pltpu.async_copy(ones_v, acc.at[dst_v.at[j]], sem, add=True))
        for d in ds:
            d.wait()
    plsc.subcore_barrier()
    _write_out(acc, out_hbm, c, s)


@functools.partial(
    pl.kernel,
    out_type=jax.ShapeDtypeStruct((NC, N, H), f32),
    mesh=_sc_mesh(),
    scratch_types=[
        pltpu.VMEM((CPT, K), jnp.int32),
        pltpu.VMEM((CPT, K), jnp.int32),
        pltpu.VMEM((K, H), f32),
        pltpu.VMEM((K, H), f32),
        pltpu.VMEM_SHARED((NPAD, H), f32),
        pltpu.SemaphoreType.DMA,
        pltpu.SemaphoreType.DMA,
        pltpu.SemaphoreType.DMA,
        pltpu.SemaphoreType.DMA,
    ],
)
def _sc_scatter128(y_hbm, src_hbm, dst_hbm, out_hbm,
                   src_v, dst_v, buf0, buf1, acc, gs0, gs1, ss0, ss1):
    c = lax.axis_index("c")
    s = lax.axis_index("s")
    tid = c * NS + s
    pltpu.sync_copy(src_hbm.at[pl.ds(tid * CPT, CPT)], src_v)
    pltpu.sync_copy(dst_hbm.at[pl.ds(tid * CPT, CPT)], dst_v)
    _fill(buf0, K, H, 0.0)
    _zero_acc_slice(buf0, acc, s * RPT, H)
    plsc.subcore_barrier()

    bufs = (buf0, buf1)
    gsem = (gs0, gs1)
    ssem = (ss0, ss1)
    gather_d = [None] * CPT
    scat_d = [None] * CPT
    gather_d[0] = pltpu.async_copy(y_hbm.at[src_v.at[0]], buf0, gs0)
    for j in range(CPT):
        b = j % 2
        nb = (j + 1) % 2
        if j + 1 < CPT:
            if j >= 1:
                scat_d[j - 1].wait()
            gather_d[j + 1] = pltpu.async_copy(
                y_hbm.at[src_v.at[j + 1]], bufs[nb], gsem[nb])
        gather_d[j].wait()
        scat_d[j] = pltpu.async_copy(bufs[b], acc.at[dst_v.at[j]], ssem[b], add=True)
    scat_d[CPT - 2].wait()
    scat_d[CPT - 1].wait()
    plsc.subcore_barrier()
    _write_out(acc, out_hbm, c, s)


def _prep_edges(edge_index):
    src = jnp.concatenate([edge_index[0], jnp.zeros((EP - E,), jnp.int32)])
    dst = jnp.concatenate([edge_index[1], jnp.full((EP - E,), N, jnp.int32)])
    return src.reshape(EP // K, K), dst.reshape(EP // K, K)


def _row_spec(w):
    return pl.BlockSpec((RB, w), lambda i: (i, 0))


def _full_spec(shape):
    nd = len(shape)
    return pl.BlockSpec(shape, lambda i: (0,) * nd)


def _tg1_body(x_ref, w_ref, cnt_ref, o_ref):
    dinv = lax.rsqrt(cnt_ref[...] + 1.0)
    o_ref[...] = jnp.dot(x_ref[...], w_ref[...], preferred_element_type=f32) * dinv


def _tg1(x, w, cnt):
    return pl.pallas_call(
        _tg1_body,
        grid=(NB,),
        in_specs=[_row_spec(D_IN), _full_spec((D_IN, H)), _row_spec(1)],
        out_specs=_row_spec(H),
        out_shape=jax.ShapeDtypeStruct((N, H), f32),
    )(x, w, cnt)


def _tg2_body(y_ref, p_ref, cnt_ref, w_ref, b_ref, o_ref):
    dinv = lax.rsqrt(cnt_ref[...] + 1.0)
    x1 = jnp.maximum((p_ref[0] + p_ref[1] + y_ref[...]) * dinv + b_ref[...], 0.0)
    o_ref[...] = jnp.dot(x1, w_ref[...], preferred_element_type=f32) * dinv


def _tg2(y, p, cnt, w, b):
    return pl.pallas_call(
        _tg2_body,
        grid=(NB,),
        in_specs=[
            _row_spec(H),
            pl.BlockSpec((NC, RB, H), lambda i: (0, i, 0)),
            _row_spec(1),
            _full_spec((H, H)),
            _full_spec((1, H)),
        ],
        out_specs=_row_spec(H),
        out_shape=jax.ShapeDtypeStruct((N, H), f32),
    )(y, p, cnt, w, b)


def _pool_body(x2, batch_ref, o_ref, acc, cacc):
    i = pl.program_id(0)

    @pl.when(i == 0)
    def _init():
        acc[...] = jnp.zeros_like(acc)
        cacc[...] = jnp.zeros_like(cacc)

    bslice = batch_ref[0]
    sel = (lax.broadcasted_iota(jnp.int32, (G, RB), 0) == bslice).astype(f32)
    acc[...] += jnp.dot(sel, x2, preferred_element_type=f32)
    cacc[...] += jnp.sum(sel, axis=1, keepdims=True)

    @pl.when(i == pl.num_programs(0) - 1)
    def _fin():
        o_ref[...] = acc[...] / jnp.maximum(cacc[...], 1.0)


def _tg3_body(y_ref, p_ref, cnt_ref, b_ref, batch_ref, o_ref, acc, cacc):
    dinv = lax.rsqrt(cnt_ref[...] + 1.0)
    x2 = jnp.maximum((p_ref[0] + p_ref[1] + y_ref[...]) * dinv + b_ref[...], 0.0)
    _pool_body(x2, batch_ref, o_ref, acc, cacc)


def _tg3(y, p, cnt, b, batch_row):
    return pl.pallas_call(
        _tg3_body,
        grid=(NB,),
        in_specs=[
            _row_spec(H),
            pl.BlockSpec((NC, RB, H), lambda i: (0, i, 0)),
            _row_spec(1),
            _full_spec((1, H)),
            pl.BlockSpec((1, 1, RB), lambda i: (i, 0, 0)),
        ],
        out_specs=_full_spec((G, H)),
        out_shape=jax.ShapeDtypeStruct((G, H), f32),
        scratch_shapes=[pltpu.VMEM((G, H), f32), pltpu.VMEM((G, 1), f32)],
    )(y, p, cnt, b, batch_row)


def _ts1_body(x_ref, w_ref, o0, o1, o2):
    res = jnp.dot(x_ref[...], w_ref[...], preferred_element_type=f32)
    o0[...] = res[:, :H]
    o1[...] = res[:, H:2 * H]
    o2[...] = res[:, 2 * H:]


def _ts1(x, wcat):
    return pl.pallas_call(
        _ts1_body,
        grid=(NB,),
        in_specs=[_row_spec(D_IN), _full_spec((D_IN, 4 * H))],
        out_specs=[_row_spec(H), _row_spec(H), _row_spec(2 * H)],
        out_shape=[
            jax.ShapeDtypeStruct((N, H), f32),
            jax.ShapeDtypeStruct((N, H), f32),
            jax.ShapeDtypeStruct((N, 2 * H), f32),
        ],
    )(x, wcat)


def _ts2_body(pa_ref, pb_ref, xr_ref, cnt_ref, w_ref, b_ref, o0, o1):
    mc = jnp.maximum(cnt_ref[...], 1.0)
    s = jnp.concatenate([pa_ref[0] + pa_ref[1], pb_ref[0] + pb_ref[1]], axis=1)
    x1 = jnp.maximum(s / mc + xr_ref[...] + b_ref[...], 0.0)
    y2 = jnp.dot(x1, w_ref[...], preferred_element_type=f32)
    o0[...] = y2[:, :H]
    o1[...] = y2[:, H:]


def _ts2(pa, pb, xr, cnt, wcat, b):
    return pl.pallas_call(
        _ts2_body,
        grid=(NB,),
        in_specs=[
            pl.BlockSpec((NC, RB, H), lambda i: (0, i, 0)),
            pl.BlockSpec((NC, RB, H), lambda i: (0, i, 0)),
            _row_spec(2 * H),
            _row_spec(1),
            _full_spec((2 * H, 2 * H)),
            _full_spec((1, 2 * H)),
        ],
        out_specs=[_row_spec(H), _row_spec(H)],
        out_shape=[
            jax.ShapeDtypeStruct((N, H), f32),
            jax.ShapeDtypeStruct((N, H), f32),
        ],
    )(pa, pb, xr, cnt, wcat, b)


def _ts3_body(xr_ref, p_ref, cnt_ref, b_ref, batch_ref, o_ref, acc, cacc):
    mc = jnp.maximum(cnt_ref[...], 1.0)
    x2 = jnp.maximum((p_ref[0] + p_ref[1]) / mc + xr_ref[...] + b_ref[...], 0.0)
    _pool_body(x2, batch_ref, o_ref, acc, cacc)


def _ts3(xr, p, cnt, b, batch_row):
    return pl.pallas_call(
        _ts3_body,
        grid=(NB,),
        in_specs=[
            _row_spec(H),
            pl.BlockSpec((NC, RB, H), lambda i: (0, i, 0)),
            _row_spec(1),
            _full_spec((1, H)),
            pl.BlockSpec((1, 1, RB), lambda i: (i, 0, 0)),
        ],
        out_specs=_full_spec((G, H)),
        out_shape=jax.ShapeDtypeStruct((G, H), f32),
        scratch_shapes=[pltpu.VMEM((G, H), f32), pltpu.VMEM((G, 1), f32)],
    )(xr, p, cnt, b, batch_row)


def kernel(grid_x, grid_edge_index, grid_batch, surf_x, surf_edge_index, surf_batch,
           Wg1, bg1, Wg2, bg2, Ws1l, Ws1r, bs1, Ws2l, Ws2r, bs2):
    src_g, dst_g = _prep_edges(grid_edge_index)
    src_s, dst_s = _prep_edges(surf_edge_index)

    cnt2 = _sc_counts(jnp.stack([dst_g, dst_s]))
    cnt_g = cnt2[0, :, :1]
    cnt_s = cnt2[1, :, :1]

    y1 = _tg1(grid_x, Wg1, cnt_g)
    p1 = _sc_scatter128(y1, src_g, dst_g)
    y2 = _tg2(y1, p1, cnt_g, Wg2, bg1.reshape(1, H))
    p2 = _sc_scatter128(y2, src_g, dst_g)
    pool_g = _tg3(y2, p2, cnt_g, bg2.reshape(1, H), grid_batch.reshape(NB, 1, RB))

    yl0, yl1, xr = _ts1(surf_x, jnp.concatenate([Ws1l, Ws1r], axis=1))
    pa = _sc_scatter128(yl0, src_s, dst_s)
    pb = _sc_scatter128(yl1, src_s, dst_s)
    yl2, xr2 = _ts2(pa, pb, xr, cnt_s,
                    jnp.concatenate([Ws2l, Ws2r], axis=1), bs1.reshape(1, 2 * H))
    pc = _sc_scatter128(yl2, src_s, dst_s)
    pool_s = _ts3(xr2, pc, cnt_s, bs2.reshape(1, H), surf_batch.reshape(NB, 1, RB))

    return jnp.concatenate([pool_g, pool_s], axis=1)
